# Initial kernel scaffold; baseline (speedup 1.0000x reference)
import jax
import jax.numpy as jnp
from jax.experimental import pallas as pl


def kernel(logits, labels):
    raise NotImplementedError("write your pallas kernel here")


# speedup vs baseline: 1.8482x; 1.8482x over previous
import jax
import jax.numpy as jnp
from jax.experimental import pallas as pl
from jax.experimental.pallas import tpu as pltpu

_B, _P = 16384, 4096
_ROWS = 256
_NB = _B // _ROWS


def _loss_block(logits_ref, labels_ref, out_ref):
    x = logits_ref[...]
    lab = labels_ref[...]
    m = jnp.max(x, axis=1, keepdims=True)
    se = jnp.sum(jnp.exp(x - m), axis=1, keepdims=True)
    lse = jnp.log(se) + m
    mask = lab != 0
    msum = jnp.sum(jnp.where(mask, x, 0.0), axis=1, keepdims=True)
    cnt = jnp.sum(mask.astype(jnp.float32), axis=1, keepdims=True)
    per_row = cnt * lse - msum
    out_ref[...] = jnp.sum(per_row).reshape(1, 1, 1)


def kernel(logits, labels):
    labels = labels.astype(jnp.int32)
    partials = pl.pallas_call(
        _loss_block,
        grid=(_NB,),
        in_specs=[
            pl.BlockSpec((_ROWS, _P), lambda i: (i, 0)),
            pl.BlockSpec((_ROWS, _P), lambda i: (i, 0)),
        ],
        out_specs=pl.BlockSpec((1, 1, 1), lambda i: (i, 0, 0)),
        out_shape=jax.ShapeDtypeStruct((_NB, 1, 1), jnp.float32),
        compiler_params=pltpu.CompilerParams(
            dimension_semantics=("parallel",),
        ),
    )(logits, labels)
    return jnp.sum(partials) / _B
